# Initial kernel scaffold; baseline (speedup 1.0000x reference)
import jax
import jax.numpy as jnp
from jax.experimental import pallas as pl


def kernel(x, router_weight, eg_w, eg_s, eu_w, eu_s, ed_w, ed_s, sg_w, sg_s, su_w, su_s, sd_w, sd_s):
    raise NotImplementedError("write your pallas kernel here")


# speedup vs baseline: 2.8204x; 2.8204x over previous
import functools

import jax
import jax.numpy as jnp
from jax.experimental import pallas as pl
from jax.experimental.pallas import tpu as pltpu

T = 2048
D = 768
I = 2048
E = 64
CAP = 128
CHUNK = 256


def _router_body(x_ref, rw_ref, tok_ref, wslot_ref, ohsum_ref, c_ref):
    x = x_ref[...]
    rw = rw_ref[...]
    logits = jax.lax.dot_general(x, rw, (((1,), (1,)), ((), ())),
                                 preferred_element_type=jnp.float32)
    eidx = jax.lax.broadcasted_iota(jnp.int32, (T, E), 1)
    big_neg = jnp.float32(-1e30)
    m1 = jnp.max(logits, axis=1, keepdims=True)
    i1 = jnp.min(jnp.where(logits == m1, eidx, E), axis=1, keepdims=True)
    oh1 = (eidx == i1)
    masked = jnp.where(oh1, big_neg, logits)
    m2 = jnp.max(masked, axis=1, keepdims=True)
    i2 = jnp.min(jnp.where(masked == m2, eidx, E), axis=1, keepdims=True)
    oh2 = (eidx == i2)
    g1 = jax.nn.sigmoid(m1 - m2)
    g2 = jax.nn.sigmoid(m2 - m1)
    oh1f = oh1.astype(jnp.float32)
    oh2f = oh2.astype(jnp.float32)
    ohsum_ref[...] = oh1f + oh2f

    tri = (jax.lax.broadcasted_iota(jnp.int32, (CHUNK, CHUNK), 0)
           > jax.lax.broadcasted_iota(jnp.int32, (CHUNK, CHUNK), 1)
           ).astype(jnp.float32)

    def body(ci, carry):
        ohc = ohsum_ref[pl.ds(ci * CHUNK, CHUNK), :]
        c_ref[pl.ds(ci * CHUNK, CHUNK), :] = carry + jax.lax.dot_general(
            tri, ohc, (((1,), (0,)), ((), ())),
            preferred_element_type=jnp.float32)
        return carry + jnp.sum(ohc, axis=0, keepdims=True)

    jax.lax.fori_loop(0, T // CHUNK, body, jnp.zeros((1, E), jnp.float32))
    cnt = c_ref[...]
    pos1 = jnp.sum(cnt * oh1f, axis=1, keepdims=True)
    pos2 = jnp.sum(cnt * oh2f, axis=1, keepdims=True)
    piota = jax.lax.broadcasted_iota(jnp.int32, (T, CAP), 1)
    ohp1 = (piota == pos1.astype(jnp.int32)).astype(jnp.float32)
    ohp2 = (piota == pos2.astype(jnp.int32)).astype(jnp.float32)
    tokv = jax.lax.broadcasted_iota(jnp.int32, (T, 1), 0).astype(jnp.float32)
    dn = (((0,), (0,)), ((), ()))
    hi = jax.lax.Precision.HIGHEST
    tok_ref[...] = (
        jax.lax.dot_general(oh1f * tokv, ohp1, dn, precision=hi,
                            preferred_element_type=jnp.float32)
        + jax.lax.dot_general(oh2f * tokv, ohp2, dn, precision=hi,
                              preferred_element_type=jnp.float32))
    wslot_ref[...] = (
        jax.lax.dot_general(oh1f * g1, ohp1, dn, precision=hi,
                            preferred_element_type=jnp.float32)
        + jax.lax.dot_general(oh2f * g2, ohp2, dn, precision=hi,
                              preferred_element_type=jnp.float32))


def _expert_body(tok_ref, w_ref, xbf_ref, eg_ref, eu_ref, ed_ref,
                 egs_ref, eus_ref, eds_ref, out_ref):
    e = pl.program_id(0)
    tok = tok_ref[0]
    w = w_ref[0]
    titer = jax.lax.broadcasted_iota(jnp.int32, (T, CAP), 0)
    oht = (titer == tok.astype(jnp.int32)).astype(jnp.float32)
    xbf = xbf_ref[...]
    h = jax.lax.dot_general(oht.astype(jnp.bfloat16), xbf,
                            (((0,), (0,)), ((), ())),
                            preferred_element_type=jnp.float32)
    hb = h.astype(jnp.bfloat16)
    dn_t = (((1,), (1,)), ((), ()))
    g = jax.lax.dot_general(hb, eg_ref[0].astype(jnp.bfloat16), dn_t,
                            preferred_element_type=jnp.float32) * egs_ref[0, 0, 0]
    u = jax.lax.dot_general(hb, eu_ref[0].astype(jnp.bfloat16), dn_t,
                            preferred_element_type=jnp.float32) * eus_ref[0, 0, 0]
    a = (g * jax.nn.sigmoid(g) * u).astype(jnp.bfloat16)
    y = jax.lax.dot_general(a, ed_ref[0].astype(jnp.bfloat16), dn_t,
                            preferred_element_type=jnp.float32) * eds_ref[0, 0, 0]
    contrib = jax.lax.dot_general(oht * w, y, (((1,), (0,)), ((), ())),
                                  preferred_element_type=jnp.float32)

    @pl.when(e == 0)
    def _init():
        out_ref[...] = contrib

    @pl.when(e != 0)
    def _acc():
        out_ref[...] += contrib


def _shared_body(xbf_ref, moe_ref, sg_ref, su_ref, sd_ref,
                 sgs_ref, sus_ref, sds_ref, out_ref,
                 sgb_ref, sub_ref, sdb_ref):
    i = pl.program_id(0)

    @pl.when(i == 0)
    def _cvt():
        sgb_ref[...] = sg_ref[...].astype(jnp.bfloat16)
        sub_ref[...] = su_ref[...].astype(jnp.bfloat16)
        sdb_ref[...] = sd_ref[...].astype(jnp.bfloat16)

    xb = xbf_ref[...]
    dn_t = (((1,), (1,)), ((), ()))
    g = jax.lax.dot_general(xb, sgb_ref[...], dn_t,
                            preferred_element_type=jnp.float32) * sgs_ref[0, 0]
    u = jax.lax.dot_general(xb, sub_ref[...], dn_t,
                            preferred_element_type=jnp.float32) * sus_ref[0, 0]
    a = (g * jax.nn.sigmoid(g) * u).astype(jnp.bfloat16)
    y = jax.lax.dot_general(a, sdb_ref[...], dn_t,
                            preferred_element_type=jnp.float32) * sds_ref[0, 0]
    out_ref[...] = y + moe_ref[...]


def kernel(x, router_weight, eg_w, eg_s, eu_w, eu_s, ed_w, ed_s,
           sg_w, sg_s, su_w, su_s, sd_w, sd_s):
    xbf = x.astype(jnp.bfloat16)

    tok, wslot = pl.pallas_call(
        _router_body,
        out_shape=[jax.ShapeDtypeStruct((E, CAP), jnp.float32),
                   jax.ShapeDtypeStruct((E, CAP), jnp.float32)],
        scratch_shapes=[pltpu.VMEM((T, E), jnp.float32),
                        pltpu.VMEM((T, E), jnp.float32)],
    )(x, router_weight)

    moe = pl.pallas_call(
        _expert_body,
        grid=(E,),
        in_specs=[
            pl.BlockSpec((1, 1, CAP), lambda e: (e, 0, 0)),
            pl.BlockSpec((1, 1, CAP), lambda e: (e, 0, 0)),
            pl.BlockSpec((T, D), lambda e: (0, 0)),
            pl.BlockSpec((1, I, D), lambda e: (e, 0, 0)),
            pl.BlockSpec((1, I, D), lambda e: (e, 0, 0)),
            pl.BlockSpec((1, D, I), lambda e: (e, 0, 0)),
            pl.BlockSpec((1, 1, 1), lambda e: (e, 0, 0)),
            pl.BlockSpec((1, 1, 1), lambda e: (e, 0, 0)),
            pl.BlockSpec((1, 1, 1), lambda e: (e, 0, 0)),
        ],
        out_specs=pl.BlockSpec((T, D), lambda e: (0, 0)),
        out_shape=jax.ShapeDtypeStruct((T, D), jnp.float32),
    )(tok.reshape(E, 1, CAP), wslot.reshape(E, 1, CAP), xbf, eg_w, eu_w, ed_w,
      eg_s.reshape(E, 1, 1), eu_s.reshape(E, 1, 1), ed_s.reshape(E, 1, 1))

    TB = 256
    out = pl.pallas_call(
        _shared_body,
        grid=(T // TB,),
        in_specs=[
            pl.BlockSpec((TB, D), lambda i: (i, 0)),
            pl.BlockSpec((TB, D), lambda i: (i, 0)),
            pl.BlockSpec((I, D), lambda i: (0, 0)),
            pl.BlockSpec((I, D), lambda i: (0, 0)),
            pl.BlockSpec((D, I), lambda i: (0, 0)),
            pl.BlockSpec((1, 1), lambda i: (0, 0)),
            pl.BlockSpec((1, 1), lambda i: (0, 0)),
            pl.BlockSpec((1, 1), lambda i: (0, 0)),
        ],
        out_specs=pl.BlockSpec((TB, D), lambda i: (i, 0)),
        out_shape=jax.ShapeDtypeStruct((T, D), jnp.float32),
        scratch_shapes=[pltpu.VMEM((I, D), jnp.bfloat16),
                        pltpu.VMEM((I, D), jnp.bfloat16),
                        pltpu.VMEM((D, I), jnp.bfloat16)],
    )(xbf, moe, sg_w, su_w, sd_w,
      sg_s.reshape(1, 1), su_s.reshape(1, 1), sd_s.reshape(1, 1))
    return out


# speedup vs baseline: 3.4275x; 1.2153x over previous
import functools

import jax
import jax.numpy as jnp
from jax import lax
from jax.experimental import pallas as pl
from jax.experimental.pallas import tpu as pltpu
from jax.experimental.pallas import tpu_sc as plsc

T = 2048
D = 768
I = 2048
E = 64
CAP = 128
NSLOT = E * CAP
CHUNK = 256
NW = 32
SC_CHUNK = 128


def _router_body(x_ref, rw_ref, disp_ref, s0_ref, s1_ref, w0_ref, w1_ref,
                 ohsum_ref, c_ref):
    x = x_ref[...]
    rw = rw_ref[...]
    logits = jax.lax.dot_general(x, rw, (((1,), (1,)), ((), ())),
                                 preferred_element_type=jnp.float32)
    eidx = jax.lax.broadcasted_iota(jnp.int32, (T, E), 1)
    big_neg = jnp.float32(-1e30)
    m1 = jnp.max(logits, axis=1, keepdims=True)
    i1 = jnp.min(jnp.where(logits == m1, eidx, E), axis=1, keepdims=True)
    oh1 = (eidx == i1)
    masked = jnp.where(oh1, big_neg, logits)
    m2 = jnp.max(masked, axis=1, keepdims=True)
    i2 = jnp.min(jnp.where(masked == m2, eidx, E), axis=1, keepdims=True)
    oh2 = (eidx == i2)
    g1 = jax.nn.sigmoid(m1 - m2)
    g2 = jax.nn.sigmoid(m2 - m1)
    oh1f = oh1.astype(jnp.float32)
    oh2f = oh2.astype(jnp.float32)
    ohsum_ref[...] = oh1f + oh2f

    tri = (jax.lax.broadcasted_iota(jnp.int32, (CHUNK, CHUNK), 0)
           > jax.lax.broadcasted_iota(jnp.int32, (CHUNK, CHUNK), 1)
           ).astype(jnp.float32)

    def body(ci, carry):
        ohc = ohsum_ref[pl.ds(ci * CHUNK, CHUNK), :]
        c_ref[pl.ds(ci * CHUNK, CHUNK), :] = carry + jax.lax.dot_general(
            tri, ohc, (((1,), (0,)), ((), ())),
            preferred_element_type=jnp.float32)
        return carry + jnp.sum(ohc, axis=0, keepdims=True)

    jax.lax.fori_loop(0, T // CHUNK, body, jnp.zeros((1, E), jnp.float32))
    cnt = c_ref[...]
    pos1 = jnp.sum(cnt * oh1f, axis=1, keepdims=True)
    pos2 = jnp.sum(cnt * oh2f, axis=1, keepdims=True)
    piota = jax.lax.broadcasted_iota(jnp.int32, (T, CAP), 1)
    ohp1 = (piota == pos1.astype(jnp.int32)).astype(jnp.float32)
    ohp2 = (piota == pos2.astype(jnp.int32)).astype(jnp.float32)
    tokv = jax.lax.broadcasted_iota(jnp.int32, (T, 1), 0).astype(jnp.float32)
    dn = (((0,), (0,)), ((), ()))
    hi = jax.lax.Precision.HIGHEST
    tokp1 = (
        jax.lax.dot_general(oh1f * (tokv + 1.0), ohp1, dn, precision=hi,
                            preferred_element_type=jnp.float32)
        + jax.lax.dot_general(oh2f * (tokv + 1.0), ohp2, dn, precision=hi,
                              preferred_element_type=jnp.float32))
    spread = jnp.bitwise_and(
        jax.lax.broadcasted_iota(jnp.int32, (E, CAP), 0) * CAP
        + jax.lax.broadcasted_iota(jnp.int32, (E, CAP), 1), T - 1)
    toki = tokp1.astype(jnp.int32)
    disp_ref[...] = jnp.where(toki == 0, spread, toki - 1)

    pos1i = pos1.astype(jnp.int32)
    pos2i = pos2.astype(jnp.int32)
    s0_ref[...] = i1 * CAP + jnp.minimum(pos1i, CAP - 1)
    s1_ref[...] = i2 * CAP + jnp.minimum(pos2i, CAP - 1)
    w0_ref[...] = jnp.where(pos1i < CAP, g1, 0.0)
    w1_ref[...] = jnp.where(pos2i < CAP, g2, 0.0)


def _expert_body(xe_ref, eg_ref, eu_ref, ed_ref,
                 egs_ref, eus_ref, eds_ref, ys_ref):
    hb = xe_ref[...].astype(jnp.bfloat16)
    dn_t = (((1,), (1,)), ((), ()))
    g = jax.lax.dot_general(hb, eg_ref[0].astype(jnp.bfloat16), dn_t,
                            preferred_element_type=jnp.float32) * egs_ref[0, 0, 0]
    u = jax.lax.dot_general(hb, eu_ref[0].astype(jnp.bfloat16), dn_t,
                            preferred_element_type=jnp.float32) * eus_ref[0, 0, 0]
    a = (g * jax.nn.sigmoid(g) * u).astype(jnp.bfloat16)
    ys_ref[...] = jax.lax.dot_general(a, ed_ref[0].astype(jnp.bfloat16), dn_t,
                                      preferred_element_type=jnp.float32
                                      ) * eds_ref[0, 0, 0]


def _make_sc_gather(n_rows):
    per_w = n_rows // NW
    chunks = per_w // SC_CHUNK
    mesh = plsc.VectorSubcoreMesh(core_axis_name="c", subcore_axis_name="s")

    @functools.partial(
        pl.kernel,
        out_type=jax.ShapeDtypeStruct((n_rows, D), jnp.float32),
        mesh=mesh,
        scratch_types=[pltpu.VMEM((SC_CHUNK,), jnp.int32),
                       pltpu.VMEM((SC_CHUNK, D), jnp.float32),
                       pltpu.SemaphoreType.DMA],
    )
    def gather(table_hbm, idx_hbm, out_hbm, idx_v, rows_v, sem):
        wid = lax.axis_index("s") * 2 + lax.axis_index("c")
        base = wid * per_w
        for c in range(chunks):
            off = base + c * SC_CHUNK
            pltpu.sync_copy(idx_hbm.at[pl.ds(off, SC_CHUNK)], idx_v)
            pltpu.async_copy(table_hbm.at[idx_v], rows_v, sem).wait()
            pltpu.sync_copy(rows_v, out_hbm.at[pl.ds(off, SC_CHUNK)])

    return gather


_sc_dispatch = _make_sc_gather(NSLOT)
_sc_combine = _make_sc_gather(2 * T)


def _shared_body(xbf_ref, y0_ref, y1_ref, w0_ref, w1_ref,
                 sg_ref, su_ref, sd_ref, sgs_ref, sus_ref, sds_ref, out_ref,
                 sgb_ref, sub_ref, sdb_ref):
    i = pl.program_id(0)

    @pl.when(i == 0)
    def _cvt():
        sgb_ref[...] = sg_ref[...].astype(jnp.bfloat16)
        sub_ref[...] = su_ref[...].astype(jnp.bfloat16)
        sdb_ref[...] = sd_ref[...].astype(jnp.bfloat16)

    xb = xbf_ref[...]
    dn_t = (((1,), (1,)), ((), ()))
    g = jax.lax.dot_general(xb, sgb_ref[...], dn_t,
                            preferred_element_type=jnp.float32) * sgs_ref[0, 0]
    u = jax.lax.dot_general(xb, sub_ref[...], dn_t,
                            preferred_element_type=jnp.float32) * sus_ref[0, 0]
    a = (g * jax.nn.sigmoid(g) * u).astype(jnp.bfloat16)
    y = jax.lax.dot_general(a, sdb_ref[...], dn_t,
                            preferred_element_type=jnp.float32) * sds_ref[0, 0]
    out_ref[...] = y + w0_ref[...] * y0_ref[...] + w1_ref[...] * y1_ref[...]


def kernel(x, router_weight, eg_w, eg_s, eu_w, eu_s, ed_w, ed_s,
           sg_w, sg_s, su_w, su_s, sd_w, sd_s):
    xbf = x.astype(jnp.bfloat16)

    disp, s0, s1, w0, w1 = pl.pallas_call(
        _router_body,
        out_shape=[jax.ShapeDtypeStruct((E, CAP), jnp.int32),
                   jax.ShapeDtypeStruct((T, 1), jnp.int32),
                   jax.ShapeDtypeStruct((T, 1), jnp.int32),
                   jax.ShapeDtypeStruct((T, 1), jnp.float32),
                   jax.ShapeDtypeStruct((T, 1), jnp.float32)],
        scratch_shapes=[pltpu.VMEM((T, E), jnp.float32),
                        pltpu.VMEM((T, E), jnp.float32)],
    )(x, router_weight)

    xe = _sc_dispatch(x, disp.reshape(NSLOT))

    ys = pl.pallas_call(
        _expert_body,
        grid=(E,),
        in_specs=[
            pl.BlockSpec((CAP, D), lambda e: (e, 0)),
            pl.BlockSpec((1, I, D), lambda e: (e, 0, 0)),
            pl.BlockSpec((1, I, D), lambda e: (e, 0, 0)),
            pl.BlockSpec((1, D, I), lambda e: (e, 0, 0)),
            pl.BlockSpec((1, 1, 1), lambda e: (e, 0, 0)),
            pl.BlockSpec((1, 1, 1), lambda e: (e, 0, 0)),
            pl.BlockSpec((1, 1, 1), lambda e: (e, 0, 0)),
        ],
        out_specs=pl.BlockSpec((CAP, D), lambda e: (e, 0)),
        out_shape=jax.ShapeDtypeStruct((NSLOT, D), jnp.float32),
    )(xe, eg_w, eu_w, ed_w,
      eg_s.reshape(E, 1, 1), eu_s.reshape(E, 1, 1), ed_s.reshape(E, 1, 1))

    slots = jnp.concatenate([s0.reshape(T), s1.reshape(T)])
    ytok = _sc_combine(ys, slots)

    TB = 256
    out = pl.pallas_call(
        _shared_body,
        grid=(T // TB,),
        in_specs=[
            pl.BlockSpec((TB, D), lambda i: (i, 0)),
            pl.BlockSpec((TB, D), lambda i: (i, 0)),
            pl.BlockSpec((TB, D), lambda i: (i + T // TB, 0)),
            pl.BlockSpec((TB, 1), lambda i: (i, 0)),
            pl.BlockSpec((TB, 1), lambda i: (i, 0)),
            pl.BlockSpec((I, D), lambda i: (0, 0)),
            pl.BlockSpec((I, D), lambda i: (0, 0)),
            pl.BlockSpec((D, I), lambda i: (0, 0)),
            pl.BlockSpec((1, 1), lambda i: (0, 0)),
            pl.BlockSpec((1, 1), lambda i: (0, 0)),
            pl.BlockSpec((1, 1), lambda i: (0, 0)),
        ],
        out_specs=pl.BlockSpec((TB, D), lambda i: (i, 0)),
        out_shape=jax.ShapeDtypeStruct((T, D), jnp.float32),
        scratch_shapes=[pltpu.VMEM((I, D), jnp.bfloat16),
                        pltpu.VMEM((I, D), jnp.bfloat16),
                        pltpu.VMEM((D, I), jnp.bfloat16)],
    )(xbf, ytok, ytok, w0, w1, sg_w, su_w, sd_w,
      sg_s.reshape(1, 1), su_s.reshape(1, 1), sd_s.reshape(1, 1))
    return out
